# Initial kernel scaffold; baseline (speedup 1.0000x reference)
import functools

import jax
import jax.numpy as jnp
from jax import lax
from jax.experimental import pallas as pl
from jax.experimental.pallas import tpu as pltpu
from jax.experimental.pallas import tpu_sc as plsc

_BATCH = 16384
_HIST = 50
_D = 64
_B = _BATCH * _HIST
_G = 128
_NUM_GROUPS = _B // _G

_info = plsc.get_sparse_core_info()
_NC, _NS = _info.num_cores, _info.num_subcores
_NW = _NC * _NS
_GPW = _NUM_GROUPS // _NW


def _embed_gather(table, idx2d):
    mesh = plsc.VectorSubcoreMesh(core_axis_name="c", subcore_axis_name="s")

    @functools.partial(
        pl.kernel,
        out_type=jax.ShapeDtypeStruct((_B, _D), jnp.float32),
        mesh=mesh,
        scratch_types=[
            pltpu.VMEM((_G,), jnp.int32),
            pltpu.VMEM((_G, _D), jnp.float32),
            pltpu.SemaphoreType.DMA,
        ],
    )
    def k(table_hbm, idx_hbm, out_hbm, idx_v, rows_v, sem):
        wid = lax.axis_index("s") * _NC + lax.axis_index("c")
        g0 = wid * _GPW

        def step(i, carry):
            g = g0 + i
            pltpu.sync_copy(idx_hbm.at[g], idx_v)
            pltpu.async_copy(table_hbm.at[idx_v], rows_v, sem).wait()
            pltpu.sync_copy(rows_v, out_hbm.at[pl.ds(g * _G, _G)])
            return carry

        lax.fori_loop(0, _GPW, step, 0)

    return k(table, idx2d)


def kernel(x, table):
    idx = x.reshape(_NUM_GROUPS, _G).astype(jnp.int32)
    out = _embed_gather(table, idx)
    return out.reshape(_BATCH, _HIST, _D)


# speedup vs baseline: 1.5824x; 1.5824x over previous
import functools

import jax
import jax.numpy as jnp
from jax import lax
from jax.experimental import pallas as pl
from jax.experimental.pallas import tpu as pltpu
from jax.experimental.pallas import tpu_sc as plsc

_BATCH = 16384
_HIST = 50
_D = 64
_B = _BATCH * _HIST
_G = 128
_NUM_GROUPS = _B // _G

_info = plsc.get_sparse_core_info()
_NC, _NS = _info.num_cores, _info.num_subcores
_NW = _NC * _NS
_GPW = _NUM_GROUPS // _NW


def _embed_gather(table, idx2d):
    mesh = plsc.VectorSubcoreMesh(core_axis_name="c", subcore_axis_name="s")

    @functools.partial(
        pl.kernel,
        out_type=jax.ShapeDtypeStruct((_B, _D), jnp.float32),
        mesh=mesh,
        scratch_types=[
            pltpu.VMEM((_G,), jnp.int32),
            pltpu.VMEM((_G, _D), jnp.float32),
            pltpu.SemaphoreType.DMA,
        ],
        compiler_params=pltpu.CompilerParams(use_tc_tiling_on_sc=False),
    )
    def k(table_hbm, idx_hbm, out_hbm, idx_v, rows_v, sem):
        wid = lax.axis_index("s") * _NC + lax.axis_index("c")
        g0 = wid * _GPW

        def step(i, carry):
            g = g0 + i
            pltpu.sync_copy(idx_hbm.at[g], idx_v)
            pltpu.async_copy(table_hbm.at[idx_v], rows_v, sem).wait()
            pltpu.sync_copy(rows_v, out_hbm.at[pl.ds(g * _G, _G)])
            return carry

        lax.fori_loop(0, _GPW, step, 0)

    return k(table, idx2d)


def kernel(x, table):
    idx = x.reshape(_NUM_GROUPS, _G).astype(jnp.int32)
    out = _embed_gather(table, idx)
    return out.reshape(_BATCH, _HIST, _D)


# speedup vs baseline: 1.8774x; 1.1864x over previous
import functools

import jax
import jax.numpy as jnp
from jax import lax
from jax.experimental import pallas as pl
from jax.experimental.pallas import tpu as pltpu
from jax.experimental.pallas import tpu_sc as plsc

_BATCH = 16384
_HIST = 50
_D = 64
_B = _BATCH * _HIST
_G = 128
_NUM_GROUPS = _B // _G

_info = plsc.get_sparse_core_info()
_NC, _NS = _info.num_cores, _info.num_subcores
_NW = _NC * _NS
_GPW = _NUM_GROUPS // _NW

_NBUF = 8
_LEAD = 4


def _embed_gather(table, idx2d):
    mesh = plsc.VectorSubcoreMesh(core_axis_name="c", subcore_axis_name="s")

    @functools.partial(
        pl.kernel,
        out_type=jax.ShapeDtypeStruct((_B, _D), jnp.float32),
        mesh=mesh,
        scratch_types=(
            [
                pltpu.VMEM((_GPW, _G), jnp.int32),
                pltpu.VMEM((_NBUF, _G, _D), jnp.float32),
            ]
            + [pltpu.SemaphoreType.DMA] * _NBUF
            + [pltpu.SemaphoreType.DMA] * _NBUF
        ),
        compiler_params=pltpu.CompilerParams(use_tc_tiling_on_sc=False),
    )
    def k(table_hbm, idx_hbm, out_hbm, idx_v, rows_v, *sems):
        gsem = sems[:_NBUF]
        ssem = sems[_NBUF:]
        wid = lax.axis_index("s") * _NC + lax.axis_index("c")
        g0 = wid * _GPW

        pltpu.sync_copy(idx_hbm.at[pl.ds(g0, _GPW)], idx_v)

        def gather(g, b):
            pltpu.async_copy(table_hbm.at[idx_v.at[g]], rows_v.at[b], gsem[b])

        def gather_wait(b):
            pltpu.make_async_copy(
                table_hbm.at[idx_v.at[0]], rows_v.at[b], gsem[b]
            ).wait()

        def scatter(g, b):
            pltpu.async_copy(
                rows_v.at[b], out_hbm.at[pl.ds((g0 + g) * _G, _G)], ssem[b]
            )

        def scatter_wait(b):
            pltpu.make_async_copy(
                rows_v.at[b], out_hbm.at[pl.ds(g0 * _G, _G)], ssem[b]
            ).wait()

        for b in range(_LEAD):
            gather(b, b)

        @pl.loop(0, _GPW, step=_NBUF)
        def _(s):
            for b in range(_NBUF):
                g = s + b
                gather_wait(b)
                scatter(g, b)
                nb = (b + _LEAD) % _NBUF
                ng = g + _LEAD

                @pl.when(ng < _GPW)
                def _():
                    @pl.when(ng >= _NBUF)
                    def _():
                        scatter_wait(nb)

                    gather(ng, nb)

        for b in range(_NBUF):
            scatter_wait(b)

    return k(table, idx2d)


def kernel(x, table):
    idx = x.reshape(_NUM_GROUPS, _G).astype(jnp.int32)
    out = _embed_gather(table, idx)
    return out.reshape(_BATCH, _HIST, _D)
